# Initial kernel scaffold; baseline (speedup 1.0000x reference)
import jax
import jax.numpy as jnp
from jax.experimental import pallas as pl


def kernel(x, w_conv, b_conv, w1, b1, gamma, beta, w2, b2):
    raise NotImplementedError("write your pallas kernel here")


# speedup vs baseline: 5.3721x; 5.3721x over previous
import functools

import jax
import jax.numpy as jnp
from jax.experimental import pallas as pl
from jax.experimental.pallas import tpu as pltpu

_LANE_PAD = 64


def _conv_pool_kernel(x_ref, wt_ref, bt_ref, o_ref, *, H, W):
    HW = H * W
    C = x_ref.shape[1]
    xp = x_ref[0]
    col = jax.lax.broadcasted_iota(jnp.int32, (C, HW), 1) % W
    zero = jnp.zeros((), jnp.bfloat16)
    slabs = []
    for di in (-1, 0, 1):
        for dj in (-1, 0, 1):
            s = _LANE_PAD + di * W + dj
            slab = xp[:, s:s + HW]
            if dj == -1:
                slab = jnp.where(col == 0, zero, slab)
            elif dj == 1:
                slab = jnp.where(col == W - 1, zero, slab)
            slabs.append(slab)
    pt = jnp.concatenate(slabs, axis=0)
    acc = jnp.dot(wt_ref[...], pt,
                  preferred_element_type=jnp.float32)
    acc = jnp.maximum(acc + bt_ref[...], 0.0)
    o_ref[...] = jnp.mean(acc, axis=1, keepdims=True)[None]


def _head_kernel(f_ref, w1_ref, b1_ref, g_ref, be_ref, w2_ref, b2_ref,
                 z_ref):
    f16 = f_ref[...].astype(jnp.bfloat16)
    h = jnp.dot(f16, w1_ref[...],
                preferred_element_type=jnp.float32) + b1_ref[...]
    mu = jnp.mean(h, axis=0, keepdims=True)
    d = h - mu
    var = jnp.mean(d * d, axis=0, keepdims=True)
    hn = d * jax.lax.rsqrt(var + 1e-5) * g_ref[...] + be_ref[...]
    hn = jnp.maximum(hn, 0.0)
    z_ref[...] = jnp.dot(hn.astype(jnp.bfloat16), w2_ref[...],
                         preferred_element_type=jnp.float32) + b2_ref[...]


def kernel(x, w_conv, b_conv, w1, b1, gamma, beta, w2, b2):
    B, C, H, W = x.shape
    HW = H * W
    K, F = w_conv.shape
    HID = w1.shape[1]
    OUT = w2.shape[1]

    x_rows = jnp.pad(x.reshape(B, C, HW).astype(jnp.bfloat16),
                     ((0, 0), (0, 0), (_LANE_PAD, _LANE_PAD)))
    w_t = jnp.transpose(
        w_conv.reshape(C, 9, F).transpose(1, 0, 2).reshape(9 * C, F)
    ).astype(jnp.bfloat16)
    b_t = b_conv.reshape(F, 1)

    feats3 = pl.pallas_call(
        functools.partial(_conv_pool_kernel, H=H, W=W),
        out_shape=jax.ShapeDtypeStruct((B, F, 1), jnp.float32),
        grid=(B,),
        in_specs=[
            pl.BlockSpec((1, C, HW + 2 * _LANE_PAD), lambda i: (i, 0, 0)),
            pl.BlockSpec((F, 9 * C), lambda i: (0, 0)),
            pl.BlockSpec((F, 1), lambda i: (0, 0)),
        ],
        out_specs=pl.BlockSpec((1, F, 1), lambda i: (i, 0, 0)),
        compiler_params=pltpu.CompilerParams(
            dimension_semantics=("parallel",)),
    )(x_rows, w_t, b_t)
    feats = feats3.reshape(B, F)

    z = pl.pallas_call(
        _head_kernel,
        out_shape=jax.ShapeDtypeStruct((B, OUT), jnp.float32),
        grid=(1,),
        in_specs=[
            pl.BlockSpec((B, F), lambda i: (0, 0)),
            pl.BlockSpec((F, HID), lambda i: (0, 0)),
            pl.BlockSpec((1, HID), lambda i: (0, 0)),
            pl.BlockSpec((1, HID), lambda i: (0, 0)),
            pl.BlockSpec((1, HID), lambda i: (0, 0)),
            pl.BlockSpec((HID, OUT), lambda i: (0, 0)),
            pl.BlockSpec((1, OUT), lambda i: (0, 0)),
        ],
        out_specs=pl.BlockSpec((B, OUT), lambda i: (0, 0)),
    )(feats, w1.astype(jnp.bfloat16), b1, gamma, beta,
      w2.astype(jnp.bfloat16), b2)
    return feats, z


# speedup vs baseline: 7.9338x; 1.4769x over previous
import functools

import jax
import jax.numpy as jnp
from jax.experimental import pallas as pl
from jax.experimental.pallas import tpu as pltpu

_LANE_PAD = 64


def _conv_pool_kernel(x_ref, wt_ref, bt_ref, o_ref, *, H, W, G):
    HW = H * W
    C = x_ref.shape[1]
    col = jax.lax.broadcasted_iota(jnp.int32, (C, HW), 1) % W
    zero = jnp.zeros((), jnp.bfloat16)
    pts = []
    for g in range(G):
        xp = x_ref[g]
        slabs = []
        for di in (-1, 0, 1):
            for dj in (-1, 0, 1):
                s = _LANE_PAD + di * W + dj
                slab = xp[:, s:s + HW]
                if dj == -1:
                    slab = jnp.where(col == 0, zero, slab)
                elif dj == 1:
                    slab = jnp.where(col == W - 1, zero, slab)
                slabs.append(slab)
        pts.append(jnp.concatenate(slabs, axis=0))
    pt = jnp.concatenate(pts, axis=1)
    acc = jnp.dot(wt_ref[...], pt,
                  preferred_element_type=jnp.float32)
    acc = jnp.maximum(acc + bt_ref[...], 0.0)
    pooled = jnp.concatenate(
        [jnp.mean(acc[:, g * HW:(g + 1) * HW], axis=1, keepdims=True)
         for g in range(G)], axis=1)
    o_ref[...] = jnp.transpose(pooled)


def _head_kernel(f_ref, w1_ref, b1_ref, g_ref, be_ref, w2_ref, b2_ref,
                 z_ref):
    f16 = f_ref[...].astype(jnp.bfloat16)
    h = jnp.dot(f16, w1_ref[...],
                preferred_element_type=jnp.float32) + b1_ref[...]
    mu = jnp.mean(h, axis=0, keepdims=True)
    d = h - mu
    var = jnp.mean(d * d, axis=0, keepdims=True)
    hn = d * jax.lax.rsqrt(var + 1e-5) * g_ref[...] + be_ref[...]
    hn = jnp.maximum(hn, 0.0)
    z_ref[...] = jnp.dot(hn.astype(jnp.bfloat16), w2_ref[...],
                         preferred_element_type=jnp.float32) + b2_ref[...]


def kernel(x, w_conv, b_conv, w1, b1, gamma, beta, w2, b2):
    B, C, H, W = x.shape
    HW = H * W
    K, F = w_conv.shape
    HID = w1.shape[1]
    OUT = w2.shape[1]

    x_rows = jnp.pad(x.reshape(B, C, HW).astype(jnp.bfloat16),
                     ((0, 0), (0, 0), (_LANE_PAD, _LANE_PAD)))
    w_t = jnp.transpose(
        w_conv.reshape(C, 9, F).transpose(1, 0, 2).reshape(9 * C, F)
    ).astype(jnp.bfloat16)
    b_t = b_conv.reshape(F, 1)

    G = 8
    assert B % G == 0
    feats = pl.pallas_call(
        functools.partial(_conv_pool_kernel, H=H, W=W, G=G),
        out_shape=jax.ShapeDtypeStruct((B, F), jnp.float32),
        grid=(B // G,),
        in_specs=[
            pl.BlockSpec((G, C, HW + 2 * _LANE_PAD), lambda i: (i, 0, 0)),
            pl.BlockSpec((F, 9 * C), lambda i: (0, 0)),
            pl.BlockSpec((F, 1), lambda i: (0, 0)),
        ],
        out_specs=pl.BlockSpec((G, F), lambda i: (i, 0)),
        compiler_params=pltpu.CompilerParams(
            dimension_semantics=("parallel",)),
    )(x_rows, w_t, b_t)

    z = pl.pallas_call(
        _head_kernel,
        out_shape=jax.ShapeDtypeStruct((B, OUT), jnp.float32),
        grid=(1,),
        in_specs=[
            pl.BlockSpec((B, F), lambda i: (0, 0)),
            pl.BlockSpec((F, HID), lambda i: (0, 0)),
            pl.BlockSpec((1, HID), lambda i: (0, 0)),
            pl.BlockSpec((1, HID), lambda i: (0, 0)),
            pl.BlockSpec((1, HID), lambda i: (0, 0)),
            pl.BlockSpec((HID, OUT), lambda i: (0, 0)),
            pl.BlockSpec((1, OUT), lambda i: (0, 0)),
        ],
        out_specs=pl.BlockSpec((B, OUT), lambda i: (0, 0)),
    )(feats, w1.astype(jnp.bfloat16), b1, gamma, beta,
      w2.astype(jnp.bfloat16), b2)
    return feats, z


# speedup vs baseline: 8.5276x; 1.0748x over previous
import functools

import jax
import jax.numpy as jnp
from jax.experimental import pallas as pl
from jax.experimental.pallas import tpu as pltpu

_LANE_PAD = 64


def _conv_pool_kernel(x_ref, wt_ref, o_ref, *, H, W, G):
    HW = H * W
    C = x_ref.shape[1]
    col = jax.lax.broadcasted_iota(jnp.int32, (C, HW), 1) % W
    zero = jnp.zeros((), jnp.bfloat16)
    pts = []
    for g in range(G):
        xp = x_ref[g]
        slabs = []
        for di in (-1, 0, 1):
            for dj in (-1, 0, 1):
                s = _LANE_PAD + di * W + dj
                slab = xp[:, s:s + HW]
                if dj == -1:
                    slab = jnp.where(col == 0, zero, slab)
                elif dj == 1:
                    slab = jnp.where(col == W - 1, zero, slab)
                slabs.append(slab)
        slabs.append(jnp.ones((8, HW), jnp.bfloat16))
        pts.append(jnp.concatenate(slabs, axis=0))
    pt = jnp.concatenate(pts, axis=1)
    acc = jnp.dot(wt_ref[...], pt,
                  preferred_element_type=jnp.float32)
    acc = jnp.maximum(acc, 0.0)
    pooled = jnp.concatenate(
        [jnp.mean(acc[:, g * HW:(g + 1) * HW], axis=1, keepdims=True)
         for g in range(G)], axis=1)
    o_ref[...] = jnp.transpose(pooled)


def _head_kernel(f_ref, w1_ref, b1_ref, g_ref, be_ref, w2_ref, b2_ref,
                 z_ref):
    f16 = f_ref[...].astype(jnp.bfloat16)
    h = jnp.dot(f16, w1_ref[...],
                preferred_element_type=jnp.float32) + b1_ref[...]
    mu = jnp.mean(h, axis=0, keepdims=True)
    d = h - mu
    var = jnp.mean(d * d, axis=0, keepdims=True)
    hn = d * jax.lax.rsqrt(var + 1e-5) * g_ref[...] + be_ref[...]
    hn = jnp.maximum(hn, 0.0)
    z_ref[...] = jnp.dot(hn.astype(jnp.bfloat16), w2_ref[...],
                         preferred_element_type=jnp.float32) + b2_ref[...]


def kernel(x, w_conv, b_conv, w1, b1, gamma, beta, w2, b2):
    B, C, H, W = x.shape
    HW = H * W
    K, F = w_conv.shape
    HID = w1.shape[1]
    OUT = w2.shape[1]

    x_rows = jnp.pad(x.reshape(B, C, HW).astype(jnp.bfloat16),
                     ((0, 0), (0, 0), (_LANE_PAD, _LANE_PAD)))
    w_t = jnp.transpose(
        w_conv.reshape(C, 9, F).transpose(1, 0, 2).reshape(9 * C, F)
    )
    b_col = b_conv.reshape(F, 1)
    b_hi = b_col.astype(jnp.bfloat16)
    b_lo = b_col - b_hi.astype(jnp.float32)
    w_t = jnp.concatenate(
        [w_t.astype(jnp.bfloat16), b_hi, b_lo.astype(jnp.bfloat16),
         jnp.zeros((F, 6), jnp.bfloat16)], axis=1)

    G = 8
    assert B % G == 0
    feats = pl.pallas_call(
        functools.partial(_conv_pool_kernel, H=H, W=W, G=G),
        out_shape=jax.ShapeDtypeStruct((B, F), jnp.float32),
        grid=(B // G,),
        in_specs=[
            pl.BlockSpec((G, C, HW + 2 * _LANE_PAD), lambda i: (i, 0, 0)),
            pl.BlockSpec((F, 9 * C + 8), lambda i: (0, 0)),
        ],
        out_specs=pl.BlockSpec((G, F), lambda i: (i, 0)),
        compiler_params=pltpu.CompilerParams(
            dimension_semantics=("parallel",)),
    )(x_rows, w_t)

    z = pl.pallas_call(
        _head_kernel,
        out_shape=jax.ShapeDtypeStruct((B, OUT), jnp.float32),
        grid=(1,),
        in_specs=[
            pl.BlockSpec((B, F), lambda i: (0, 0)),
            pl.BlockSpec((F, HID), lambda i: (0, 0)),
            pl.BlockSpec((1, HID), lambda i: (0, 0)),
            pl.BlockSpec((1, HID), lambda i: (0, 0)),
            pl.BlockSpec((1, HID), lambda i: (0, 0)),
            pl.BlockSpec((HID, OUT), lambda i: (0, 0)),
            pl.BlockSpec((1, OUT), lambda i: (0, 0)),
        ],
        out_specs=pl.BlockSpec((B, OUT), lambda i: (0, 0)),
    )(feats, w1.astype(jnp.bfloat16), b1, gamma, beta,
      w2.astype(jnp.bfloat16), b2)
    return feats, z


# speedup vs baseline: 10.7865x; 1.2649x over previous
import functools

import jax
import jax.numpy as jnp
from jax.experimental import pallas as pl
from jax.experimental.pallas import tpu as pltpu


def _conv_pool_kernel(x_ref, wt_ref, o_ref, *, H, W, G):
    HW = H * W
    C = x_ref.shape[1]
    idx = jax.lax.broadcasted_iota(jnp.int32, (C, HW), 1)
    colj = idx % W
    zero = jnp.zeros((), jnp.bfloat16)
    pts = []
    for g in range(G):
        xr = x_ref[g].astype(jnp.bfloat16)
        slabs = []
        for di in (-1, 0, 1):
            for dj in (-1, 0, 1):
                s = di * W + dj
                slab = jnp.roll(xr, -s, axis=1) if s else xr
                valid = None
                if di == -1:
                    valid = idx >= -s
                elif di == 1:
                    valid = idx < HW - s
                if dj == -1:
                    wmask = colj != 0
                    valid = wmask if valid is None else (valid & wmask)
                elif dj == 1:
                    wmask = colj != W - 1
                    valid = wmask if valid is None else (valid & wmask)
                if valid is not None:
                    slab = jnp.where(valid, slab, zero)
                slabs.append(slab)
        slabs.append(jnp.ones((8, HW), jnp.bfloat16))
        pts.append(jnp.concatenate(slabs, axis=0))
    pt = jnp.concatenate(pts, axis=1)
    acc = jnp.dot(wt_ref[...], pt,
                  preferred_element_type=jnp.float32)
    acc = jnp.maximum(acc, 0.0)
    pooled = jnp.concatenate(
        [jnp.mean(acc[:, g * HW:(g + 1) * HW], axis=1, keepdims=True)
         for g in range(G)], axis=1)
    o_ref[...] = jnp.transpose(pooled)


def _head_kernel(f_ref, w1_ref, b1_ref, g_ref, be_ref, w2_ref, b2_ref,
                 z_ref):
    f16 = f_ref[...].astype(jnp.bfloat16)
    h = jnp.dot(f16, w1_ref[...],
                preferred_element_type=jnp.float32) + b1_ref[...]
    mu = jnp.mean(h, axis=0, keepdims=True)
    d = h - mu
    var = jnp.mean(d * d, axis=0, keepdims=True)
    hn = d * jax.lax.rsqrt(var + 1e-5) * g_ref[...] + be_ref[...]
    hn = jnp.maximum(hn, 0.0)
    z_ref[...] = jnp.dot(hn.astype(jnp.bfloat16), w2_ref[...],
                         preferred_element_type=jnp.float32) + b2_ref[...]


def kernel(x, w_conv, b_conv, w1, b1, gamma, beta, w2, b2):
    B, C, H, W = x.shape
    HW = H * W
    K, F = w_conv.shape
    HID = w1.shape[1]
    OUT = w2.shape[1]

    x_rows = x.reshape(B, C, HW)
    w_t = jnp.transpose(
        w_conv.reshape(C, 9, F).transpose(1, 0, 2).reshape(9 * C, F)
    )
    b_col = b_conv.reshape(F, 1)
    b_hi = b_col.astype(jnp.bfloat16)
    b_lo = b_col - b_hi.astype(jnp.float32)
    w_t = jnp.concatenate(
        [w_t.astype(jnp.bfloat16), b_hi, b_lo.astype(jnp.bfloat16),
         jnp.zeros((F, 6), jnp.bfloat16)], axis=1)

    G = 8
    assert B % G == 0
    feats = pl.pallas_call(
        functools.partial(_conv_pool_kernel, H=H, W=W, G=G),
        out_shape=jax.ShapeDtypeStruct((B, F), jnp.float32),
        grid=(B // G,),
        in_specs=[
            pl.BlockSpec((G, C, HW), lambda i: (i, 0, 0)),
            pl.BlockSpec((F, 9 * C + 8), lambda i: (0, 0)),
        ],
        out_specs=pl.BlockSpec((G, F), lambda i: (i, 0)),
        compiler_params=pltpu.CompilerParams(
            dimension_semantics=("parallel",)),
    )(x_rows, w_t)

    z = pl.pallas_call(
        _head_kernel,
        out_shape=jax.ShapeDtypeStruct((B, OUT), jnp.float32),
        grid=(1,),
        in_specs=[
            pl.BlockSpec((B, F), lambda i: (0, 0)),
            pl.BlockSpec((F, HID), lambda i: (0, 0)),
            pl.BlockSpec((1, HID), lambda i: (0, 0)),
            pl.BlockSpec((1, HID), lambda i: (0, 0)),
            pl.BlockSpec((1, HID), lambda i: (0, 0)),
            pl.BlockSpec((HID, OUT), lambda i: (0, 0)),
            pl.BlockSpec((1, OUT), lambda i: (0, 0)),
        ],
        out_specs=pl.BlockSpec((B, OUT), lambda i: (0, 0)),
    )(feats, w1.astype(jnp.bfloat16), b1, gamma, beta,
      w2.astype(jnp.bfloat16), b2)
    return feats, z


# speedup vs baseline: 11.0215x; 1.0218x over previous
import functools

import jax
import jax.numpy as jnp
from jax.experimental import pallas as pl
from jax.experimental.pallas import tpu as pltpu


def _conv_pool_kernel(x_ref, wt_ref, o_ref, *, H, W, G):
    HW = H * W
    C = x_ref.shape[1]
    idx = jax.lax.broadcasted_iota(jnp.int32, (C, HW), 1)
    colj = idx % W
    zero = jnp.zeros((), jnp.bfloat16)
    pts = []
    for g in range(G):
        xr = x_ref[g].astype(jnp.bfloat16)
        slabs = []
        for di in (-1, 0, 1):
            for dj in (-1, 0, 1):
                s = di * W + dj
                slab = jnp.roll(xr, -s, axis=1) if s else xr
                valid = None
                if di == -1:
                    valid = idx >= -s
                elif di == 1:
                    valid = idx < HW - s
                if dj == -1:
                    wmask = colj != 0
                    valid = wmask if valid is None else (valid & wmask)
                elif dj == 1:
                    wmask = colj != W - 1
                    valid = wmask if valid is None else (valid & wmask)
                if valid is not None:
                    slab = jnp.where(valid, slab, zero)
                slabs.append(slab)
        slabs.append(jnp.ones((8, HW), jnp.bfloat16))
        pts.append(jnp.concatenate(slabs, axis=0))
    pt = jnp.concatenate(pts, axis=1)
    acc = jnp.dot(wt_ref[...], pt,
                  preferred_element_type=jnp.float32)
    acc = jnp.maximum(acc, 0.0)
    pooled = jnp.concatenate(
        [jnp.mean(acc[:, g * HW:(g + 1) * HW], axis=1, keepdims=True)
         for g in range(G)], axis=1)
    o_ref[...] = jnp.transpose(pooled)


def _head_kernel(f_ref, w1_ref, b1_ref, g_ref, be_ref, w2_ref, b2_ref,
                 z_ref):
    f16 = f_ref[...].astype(jnp.bfloat16)
    h = jnp.dot(f16, w1_ref[...],
                preferred_element_type=jnp.float32) + b1_ref[...]
    mu = jnp.mean(h, axis=0, keepdims=True)
    d = h - mu
    var = jnp.mean(d * d, axis=0, keepdims=True)
    hn = d * jax.lax.rsqrt(var + 1e-5) * g_ref[...] + be_ref[...]
    hn = jnp.maximum(hn, 0.0)
    z_ref[...] = jnp.dot(hn.astype(jnp.bfloat16), w2_ref[...],
                         preferred_element_type=jnp.float32) + b2_ref[...]


def kernel(x, w_conv, b_conv, w1, b1, gamma, beta, w2, b2):
    B, C, H, W = x.shape
    HW = H * W
    K, F = w_conv.shape
    HID = w1.shape[1]
    OUT = w2.shape[1]

    x_rows = x.reshape(B, C, HW)
    w_t = jnp.transpose(
        w_conv.reshape(C, 9, F).transpose(1, 0, 2).reshape(9 * C, F)
    )
    b_col = b_conv.reshape(F, 1)
    b_hi = b_col.astype(jnp.bfloat16)
    b_lo = b_col - b_hi.astype(jnp.float32)
    w_t = jnp.concatenate(
        [w_t.astype(jnp.bfloat16), b_hi, b_lo.astype(jnp.bfloat16),
         jnp.zeros((F, 6), jnp.bfloat16)], axis=1)

    G = 16
    assert B % G == 0
    feats = pl.pallas_call(
        functools.partial(_conv_pool_kernel, H=H, W=W, G=G),
        out_shape=jax.ShapeDtypeStruct((B, F), jnp.float32),
        grid=(B // G,),
        in_specs=[
            pl.BlockSpec((G, C, HW), lambda i: (i, 0, 0)),
            pl.BlockSpec((F, 9 * C + 8), lambda i: (0, 0)),
        ],
        out_specs=pl.BlockSpec((G, F), lambda i: (i, 0)),
        compiler_params=pltpu.CompilerParams(
            dimension_semantics=("arbitrary",)),
    )(x_rows, w_t)

    z = pl.pallas_call(
        _head_kernel,
        out_shape=jax.ShapeDtypeStruct((B, OUT), jnp.float32),
        grid=(1,),
        in_specs=[
            pl.BlockSpec((B, F), lambda i: (0, 0)),
            pl.BlockSpec((F, HID), lambda i: (0, 0)),
            pl.BlockSpec((1, HID), lambda i: (0, 0)),
            pl.BlockSpec((1, HID), lambda i: (0, 0)),
            pl.BlockSpec((1, HID), lambda i: (0, 0)),
            pl.BlockSpec((HID, OUT), lambda i: (0, 0)),
            pl.BlockSpec((1, OUT), lambda i: (0, 0)),
        ],
        out_specs=pl.BlockSpec((B, OUT), lambda i: (0, 0)),
    )(feats, w1.astype(jnp.bfloat16), b1, gamma, beta,
      w2.astype(jnp.bfloat16), b2)
    return feats, z


# speedup vs baseline: 11.0610x; 1.0036x over previous
import functools

import jax
import jax.numpy as jnp
from jax.experimental import pallas as pl
from jax.experimental.pallas import tpu as pltpu


def _conv_pool_kernel(x_ref, wt_ref, o_ref, *, H, W, G):
    HW = H * W
    C = x_ref.shape[1]
    idx = jax.lax.broadcasted_iota(jnp.int32, (C, HW), 1)
    colj = idx % W
    zero = jnp.zeros((), jnp.bfloat16)
    pts = []
    for g in range(G):
        xr = x_ref[g].astype(jnp.bfloat16)
        slabs = []
        for di in (-1, 0, 1):
            for dj in (-1, 0, 1):
                s = di * W + dj
                slab = jnp.roll(xr, -s, axis=1) if s else xr
                valid = None
                if di == -1:
                    valid = idx >= -s
                elif di == 1:
                    valid = idx < HW - s
                if dj == -1:
                    wmask = colj != 0
                    valid = wmask if valid is None else (valid & wmask)
                elif dj == 1:
                    wmask = colj != W - 1
                    valid = wmask if valid is None else (valid & wmask)
                if valid is not None:
                    slab = jnp.where(valid, slab, zero)
                slabs.append(slab)
        slabs.append(jnp.ones((8, HW), jnp.bfloat16))
        pt = jnp.concatenate(slabs, axis=0)
        acc = jnp.dot(wt_ref[...], pt,
                      preferred_element_type=jnp.float32)
        acc = jnp.maximum(acc, 0.0)
        pts.append(jnp.mean(acc, axis=1, keepdims=True))
    o_ref[...] = jnp.transpose(jnp.concatenate(pts, axis=1))


def _head_kernel(f_ref, w1_ref, b1_ref, g_ref, be_ref, w2_ref, b2_ref,
                 z_ref):
    f16 = f_ref[...].astype(jnp.bfloat16)
    h = jnp.dot(f16, w1_ref[...],
                preferred_element_type=jnp.float32) + b1_ref[...]
    mu = jnp.mean(h, axis=0, keepdims=True)
    d = h - mu
    var = jnp.mean(d * d, axis=0, keepdims=True)
    hn = d * jax.lax.rsqrt(var + 1e-5) * g_ref[...] + be_ref[...]
    hn = jnp.maximum(hn, 0.0)
    z_ref[...] = jnp.dot(hn.astype(jnp.bfloat16), w2_ref[...],
                         preferred_element_type=jnp.float32) + b2_ref[...]


def kernel(x, w_conv, b_conv, w1, b1, gamma, beta, w2, b2):
    B, C, H, W = x.shape
    HW = H * W
    K, F = w_conv.shape
    HID = w1.shape[1]
    OUT = w2.shape[1]

    x_rows = x.reshape(B, C, HW)
    w_t = jnp.transpose(
        w_conv.reshape(C, 9, F).transpose(1, 0, 2).reshape(9 * C, F)
    )
    b_col = b_conv.reshape(F, 1)
    b_hi = b_col.astype(jnp.bfloat16)
    b_lo = b_col - b_hi.astype(jnp.float32)
    w_t = jnp.concatenate(
        [w_t.astype(jnp.bfloat16), b_hi, b_lo.astype(jnp.bfloat16),
         jnp.zeros((F, 6), jnp.bfloat16)], axis=1)

    G = 16
    assert B % G == 0
    feats = pl.pallas_call(
        functools.partial(_conv_pool_kernel, H=H, W=W, G=G),
        out_shape=jax.ShapeDtypeStruct((B, F), jnp.float32),
        grid=(B // G,),
        in_specs=[
            pl.BlockSpec((G, C, HW), lambda i: (i, 0, 0)),
            pl.BlockSpec((F, 9 * C + 8), lambda i: (0, 0)),
        ],
        out_specs=pl.BlockSpec((G, F), lambda i: (i, 0)),
        compiler_params=pltpu.CompilerParams(
            dimension_semantics=("arbitrary",)),
    )(x_rows, w_t)

    z = pl.pallas_call(
        _head_kernel,
        out_shape=jax.ShapeDtypeStruct((B, OUT), jnp.float32),
        grid=(1,),
        in_specs=[
            pl.BlockSpec((B, F), lambda i: (0, 0)),
            pl.BlockSpec((F, HID), lambda i: (0, 0)),
            pl.BlockSpec((1, HID), lambda i: (0, 0)),
            pl.BlockSpec((1, HID), lambda i: (0, 0)),
            pl.BlockSpec((1, HID), lambda i: (0, 0)),
            pl.BlockSpec((HID, OUT), lambda i: (0, 0)),
            pl.BlockSpec((1, OUT), lambda i: (0, 0)),
        ],
        out_specs=pl.BlockSpec((B, OUT), lambda i: (0, 0)),
    )(feats, w1.astype(jnp.bfloat16), b1, gamma, beta,
      w2.astype(jnp.bfloat16), b2)
    return feats, z
